# Initial kernel scaffold; baseline (speedup 1.0000x reference)
import jax
import jax.numpy as jnp
from jax.experimental import pallas as pl


def kernel(tokenized_text, embedded_text, lora_up, lora_down, bias):
    raise NotImplementedError("write your pallas kernel here")


# speedup vs baseline: 1.3162x; 1.3162x over previous
import functools

import jax
import jax.numpy as jnp
from jax import lax
from jax.experimental import pallas as pl
from jax.experimental.pallas import tpu as pltpu
from jax.experimental.pallas import tpu_sc as plsc

_L = 25
_PH = 49408
_D = 768
_SCALE = 1.0


def _pe_body(up_ref, down_ref, bias_ref, out_ref):
    out_ref[...] = (
        jnp.dot(up_ref[...], down_ref[...], preferred_element_type=jnp.float32)
        * _SCALE
        + bias_ref[...]
    )


def _sc_body(nc, b_dim, n_pad, tok_hbm, emb_hbm, pe_hbm, out_hbm,
             tok_v, emb_v, pe_v):
    cid = lax.axis_index("c")
    sid = lax.axis_index("s")
    wid = sid * nc + cid
    b = wid // 2
    half = wid % 2
    l_lo = half * 13
    l_hi = jnp.where(half == 0, 13, _L)

    pltpu.sync_copy(tok_hbm.at[b], tok_v)
    pltpu.sync_copy(emb_hbm.at[b], emb_v)
    pltpu.sync_copy(pe_hbm, pe_v)

    iota = lax.iota(jnp.int32, 16)
    n_seq = emb_v.shape[0]

    pos = jnp.int32(-1)
    for c in range(n_pad // 16):
        chunk = tok_v[pl.ds(c * 16, 16)]
        for j in range(16):
            pos = jnp.where(chunk[j] == _PH, c * 16 + j, pos)
    row = jnp.clip(pos, 0, n_seq - 1)

    def body(l, carry):
        @pl.when(pos >= 0)
        def _():
            for j in range(_D // 16):
                emb_v[row, pl.ds(j * 16, 16)] = pe_v[l, pl.ds(j * 16, 16)]

        pltpu.sync_copy(emb_v, out_hbm.at[b * _L + l])
        return carry

    lax.fori_loop(l_lo, l_hi, body, 0)


def kernel(tokenized_text, embedded_text, lora_up, lora_down, bias):
    b_dim, n = tokenized_text.shape
    n_pad = ((n + 15) // 16) * 16

    pe = pl.pallas_call(
        _pe_body,
        out_shape=jax.ShapeDtypeStruct((_L, _D), jnp.float32),
    )(lora_up, lora_down, bias.reshape(1, _D))

    tok = jnp.pad(tokenized_text, ((0, 0), (0, n_pad - n)))

    info = plsc.get_sparse_core_info()
    nc = info.num_cores
    mesh = plsc.VectorSubcoreMesh(core_axis_name="c", subcore_axis_name="s")

    out = pl.kernel(
        functools.partial(_sc_body, nc, b_dim, n_pad),
        out_type=jax.ShapeDtypeStruct((b_dim * _L, n, _D), jnp.float32),
        mesh=mesh,
        scratch_types=[
            pltpu.VMEM((n_pad,), jnp.int32),
            pltpu.VMEM((n, _D), jnp.float32),
            pltpu.VMEM((_L, _D), jnp.float32),
        ],
    )(tok, embedded_text, pe)
    return out


# speedup vs baseline: 1.3458x; 1.0225x over previous
import functools

import jax
import jax.numpy as jnp
from jax import lax
from jax.experimental import pallas as pl
from jax.experimental.pallas import tpu as pltpu
from jax.experimental.pallas import tpu_sc as plsc

_L = 25
_PH = 49408
_D = 768
_SCALE = 1.0
_LMAX = 13


def _pe_body(up_ref, down_ref, bias_ref, out_ref):
    out_ref[...] = (
        jnp.dot(up_ref[...], down_ref[...], preferred_element_type=jnp.float32)
        * _SCALE
        + bias_ref[...]
    )


def _sc_body(nc, n_pad, tok_hbm, emb_hbm, pe_hbm, out_hbm,
             tok_v, emb_v, pe_v, read_sem, big_sem, small_sem):
    cid = lax.axis_index("c")
    sid = lax.axis_index("s")
    wid = sid * nc + cid
    b = wid // 2
    half = wid % 2
    l_lo = half * _LMAX
    l_hi = jnp.where(half == 0, _LMAX, _L)
    n_seq = emb_v.shape[0]

    pltpu.make_async_copy(tok_hbm.at[b], tok_v, read_sem).start()
    pltpu.make_async_copy(emb_hbm.at[b], emb_v, read_sem).start()
    pltpu.make_async_copy(pe_hbm, pe_v, read_sem).start()
    pltpu.make_async_copy(tok_hbm.at[b], tok_v, read_sem).wait()
    pltpu.make_async_copy(emb_hbm.at[b], emb_v, read_sem).wait()
    pltpu.make_async_copy(pe_hbm, pe_v, read_sem).wait()

    for l_off in range(_LMAX):
        l = l_lo + l_off

        @pl.when(l < l_hi)
        def _():
            pltpu.make_async_copy(emb_v, out_hbm.at[b * _L + l], big_sem).start()

    pos = jnp.int32(-1)
    for c in range(n_pad // 16):
        chunk = tok_v[pl.ds(c * 16, 16)]
        for j in range(16):
            pos = jnp.where(chunk[j] == _PH, c * 16 + j, pos)
    row = jnp.clip(pos, 0, n_seq - 1)

    for l_off in range(_LMAX):
        l = l_lo + l_off

        @pl.when(l < l_hi)
        def _():
            pltpu.make_async_copy(emb_v, out_hbm.at[b * _L + l], big_sem).wait()

    for l_off in range(_LMAX):
        l = l_lo + l_off

        @pl.when((l < l_hi) & (pos >= 0))
        def _():
            pltpu.make_async_copy(
                pe_v.at[l], out_hbm.at[b * _L + l, row], small_sem).start()

    for l_off in range(_LMAX):
        l = l_lo + l_off

        @pl.when((l < l_hi) & (pos >= 0))
        def _():
            pltpu.make_async_copy(
                pe_v.at[l], out_hbm.at[b * _L + l, row], small_sem).wait()


def kernel(tokenized_text, embedded_text, lora_up, lora_down, bias):
    b_dim, n = tokenized_text.shape
    n_pad = ((n + 15) // 16) * 16

    pe = pl.pallas_call(
        _pe_body,
        out_shape=jax.ShapeDtypeStruct((_L, _D), jnp.float32),
    )(lora_up, lora_down, bias.reshape(1, _D))

    tok = jnp.pad(tokenized_text, ((0, 0), (0, n_pad - n)))

    info = plsc.get_sparse_core_info()
    nc = info.num_cores
    mesh = plsc.VectorSubcoreMesh(core_axis_name="c", subcore_axis_name="s")

    out = pl.kernel(
        functools.partial(_sc_body, nc, n_pad),
        out_type=jax.ShapeDtypeStruct((b_dim * _L, n, _D), jnp.float32),
        mesh=mesh,
        scratch_types=[
            pltpu.VMEM((n_pad,), jnp.int32),
            pltpu.VMEM((n, _D), jnp.float32),
            pltpu.VMEM((_L, _D), jnp.float32),
            pltpu.SemaphoreType.DMA,
            pltpu.SemaphoreType.DMA,
            pltpu.SemaphoreType.DMA,
        ],
    )(tok, embedded_text, pe)
    return out
